# Initial kernel scaffold; baseline (speedup 1.0000x reference)
import jax
import jax.numpy as jnp
from jax.experimental import pallas as pl


def kernel(logits):
    raise NotImplementedError("write your pallas kernel here")


# speedup vs baseline: 7.3787x; 7.3787x over previous
import jax
import jax.numpy as jnp
from jax.experimental import pallas as pl
from jax.experimental.pallas import tpu as pltpu

B, K, N = 64, 16, 32768
TEMP_INV = 2.0

KEY0 = 928981903
KEY1 = 3453687069


def _threefry2x32(x0, x1):
    k0 = jnp.uint32(KEY0)
    k1 = jnp.uint32(KEY1)
    k2 = jnp.uint32(KEY0 ^ KEY1 ^ 0x1BD11BDA)
    ks = (k0, k1, k2)
    rots = ((13, 15, 26, 6), (17, 29, 16, 24))

    def rotl(x, r):
        return (x << jnp.uint32(r)) | (x >> jnp.uint32(32 - r))

    x0 = x0 + k0
    x1 = x1 + k1
    for g in range(5):
        for r in rots[g % 2]:
            x0 = x0 + x1
            x1 = rotl(x1, r)
            x1 = x1 ^ x0
        x0 = x0 + ks[(g + 1) % 3]
        x1 = x1 + ks[(g + 2) % 3] + jnp.uint32(g + 1)
    return x0, x1


def _kernel(logits_ref, dsamples_ref, csamples_ref):
    b = pl.program_id(0)
    l_row = logits_ref[0]

    k_iota = jax.lax.broadcasted_iota(jnp.uint32, (K, N), 0)
    n_iota = jax.lax.broadcasted_iota(jnp.uint32, (K, N), 1)
    base = (jnp.uint32(b) * jnp.uint32(K) + k_iota) * jnp.uint32(N) + n_iota
    o0, o1 = _threefry2x32(jnp.zeros((K, N), jnp.uint32), base)
    bits = o0 ^ o1

    mant = (bits >> jnp.uint32(9)) | jnp.uint32(0x3F800000)
    u = jax.lax.bitcast_convert_type(mant, jnp.float32) - jnp.float32(1.0)
    eps = jnp.finfo(jnp.float32).eps
    u = jnp.clip(u, eps, 1.0 - eps)
    gumbel = -jnp.log(-jnp.log(u))
    nl = (gumbel + l_row) * TEMP_INV
    m = jnp.max(nl, axis=1, keepdims=True)
    e = jnp.exp(nl - m)
    s = jnp.sum(e, axis=1, keepdims=True)
    samples = e / s
    csamples_ref[...] = jnp.max(samples, axis=0, keepdims=True)[None]

    lb = jax.lax.bitcast_convert_type(l_row + 0.0, jnp.uint32)
    neg = (lb >> jnp.uint32(31)) == jnp.uint32(1)
    ukey = jnp.where(neg, ~lb, lb | jnp.uint32(0x80000000))
    t = jnp.uint32(0)
    for bit in range(31, -1, -1):
        cand = t | jnp.uint32(1 << bit)
        cnt = jnp.sum((ukey >= cand).astype(jnp.int32))
        t = jnp.where(cnt >= K, cand, t)
    dsamples_ref[...] = ((ukey >= t).astype(jnp.float32))[None]


def kernel(logits):
    spec = pl.BlockSpec((1, 1, N), lambda b: (b, 0, 0))
    dsamples, csamples = pl.pallas_call(
        _kernel,
        grid=(B,),
        in_specs=[spec],
        out_specs=[spec, spec],
        out_shape=[
            jax.ShapeDtypeStruct((B, 1, N), jnp.float32),
            jax.ShapeDtypeStruct((B, 1, N), jnp.float32),
        ],
        compiler_params=pltpu.CompilerParams(
            dimension_semantics=("parallel",),
        ),
    )(logits)
    return dsamples.reshape(B, N), csamples.reshape(B, N)


# speedup vs baseline: 9.7992x; 1.3280x over previous
import jax
import jax.numpy as jnp
from jax.experimental import pallas as pl
from jax.experimental.pallas import tpu as pltpu

B, K, N = 64, 16, 32768
RB = 8

KEY0 = 928981903
KEY1 = 3453687069


def _threefry2x32(x0, x1):
    k0 = jnp.uint32(KEY0)
    k1 = jnp.uint32(KEY1)
    k2 = jnp.uint32(KEY0 ^ KEY1 ^ 0x1BD11BDA)
    ks = (k0, k1, k2)
    rots = ((13, 15, 26, 6), (17, 29, 16, 24))

    def rotl(x, r):
        return (x << jnp.uint32(r)) | (x >> jnp.uint32(32 - r))

    x0 = x0 + k0
    x1 = x1 + k1
    for g in range(5):
        for r in rots[g % 2]:
            x0 = x0 + x1
            x1 = rotl(x1, r)
            x1 = x1 ^ x0
        x0 = x0 + ks[(g + 1) % 3]
        x1 = x1 + ks[(g + 2) % 3] + jnp.uint32(g + 1)
    return x0, x1


def _csamples_kernel(logits_ref, csamples_ref):
    b = pl.program_id(0)
    l_row = logits_ref[0]

    k_iota = jax.lax.broadcasted_iota(jnp.uint32, (K, N), 0)
    n_iota = jax.lax.broadcasted_iota(jnp.uint32, (K, N), 1)
    base = (jnp.uint32(b) * jnp.uint32(K) + k_iota) * jnp.uint32(N) + n_iota
    o0, o1 = _threefry2x32(jnp.zeros((K, N), jnp.uint32), base)
    bits = o0 ^ o1

    mant = (bits >> jnp.uint32(9)) | jnp.uint32(0x3F800000)
    u = jax.lax.bitcast_convert_type(mant, jnp.float32) - jnp.float32(1.0)
    eps = jnp.finfo(jnp.float32).eps
    u = jnp.clip(u, eps, 1.0 - eps)
    w = -jnp.log(u)

    e_l = jnp.exp(l_row - jnp.max(l_row))
    q = e_l / w
    m = jnp.max(q, axis=1, keepdims=True)
    t = q * (1.0 / m)
    t2 = t * t
    s = jnp.sum(t2, axis=1, keepdims=True)
    samples = t2 / s
    csamples_ref[...] = jnp.max(samples, axis=0, keepdims=True)[None]


def _dsamples_kernel(logits_ref, dsamples_ref):
    l_rows = logits_ref[0]

    lb = jax.lax.bitcast_convert_type(l_rows + 0.0, jnp.uint32)
    neg = (lb >> jnp.uint32(31)) == jnp.uint32(1)
    ukey = jnp.where(neg, ~lb, lb | jnp.uint32(0x80000000))
    t = jnp.zeros((RB, 1), jnp.uint32)
    for bit in range(31, -1, -1):
        cand = t | jnp.uint32(1 << bit)
        cnt = jnp.sum((ukey >= cand).astype(jnp.int32), axis=1, keepdims=True)
        t = jnp.where(cnt >= K, cand, t)
    dsamples_ref[...] = ((ukey >= t).astype(jnp.float32))[None]


def kernel(logits):
    spec_row = pl.BlockSpec((1, 1, N), lambda b: (b, 0, 0))
    csamples = pl.pallas_call(
        _csamples_kernel,
        grid=(B,),
        in_specs=[spec_row],
        out_specs=spec_row,
        out_shape=jax.ShapeDtypeStruct((B, 1, N), jnp.float32),
        compiler_params=pltpu.CompilerParams(
            dimension_semantics=("parallel",),
        ),
    )(logits)

    l_blocks = logits.reshape(B // RB, RB, N)
    spec_blk = pl.BlockSpec((1, RB, N), lambda b: (b, 0, 0))
    dsamples = pl.pallas_call(
        _dsamples_kernel,
        grid=(B // RB,),
        in_specs=[spec_blk],
        out_specs=spec_blk,
        out_shape=jax.ShapeDtypeStruct((B // RB, RB, N), jnp.float32),
        compiler_params=pltpu.CompilerParams(
            dimension_semantics=("parallel",),
        ),
    )(l_blocks)
    return dsamples.reshape(B, N), csamples.reshape(B, N)


# speedup vs baseline: 15.1657x; 1.5476x over previous
import jax
import jax.numpy as jnp
from jax.experimental import pallas as pl
from jax.experimental.pallas import tpu as pltpu

B, K, N = 64, 16, 32768
SUB, LANE = 8, 4096
CH = 1024
NCH = LANE // CH
RB = 16

KEY0 = 928981903
KEY1 = 3453687069


def _threefry_keyed(x0, x1):
    k0 = jnp.uint32(KEY0)
    k1 = jnp.uint32(KEY1)
    k2 = jnp.uint32(KEY0 ^ KEY1 ^ 0x1BD11BDA)
    ks = (k0, k1, k2)
    rots = ((13, 15, 26, 6), (17, 29, 16, 24))

    def rotl(x, r):
        return (x << jnp.uint32(r)) | (x >> jnp.uint32(32 - r))

    for g in range(5):
        for r in rots[g % 2]:
            x0 = x0 + x1
            x1 = rotl(x1, r)
            x1 = x1 ^ x0
        x0 = x0 + ks[(g + 1) % 3]
        x1 = x1 + ks[(g + 2) % 3] + jnp.uint32(g + 1)
    return x0, x1


def _csamples_kernel(logits_ref, csamples_ref, e8_ref, q2a_ref, q2b_ref):
    b = pl.program_id(0)

    lmax = jnp.float32(-jnp.inf)
    for c in range(NCH):
        sl = pl.ds(c * CH, CH)
        lmax = jnp.maximum(lmax, jnp.max(logits_ref[0, :, sl]))
    zero = jnp.zeros((SUB, CH), jnp.float32)
    for c in range(NCH):
        sl = pl.ds(c * CH, CH)
        e8_ref[:, sl] = jnp.exp(logits_ref[0, :, sl] - lmax)
        csamples_ref[0, :, sl] = zero
        q2b_ref[:, sl] = zero

    r_iota = jax.lax.broadcasted_iota(jnp.uint32, (SUB, CH), 0)
    c_iota = jax.lax.broadcasted_iota(jnp.uint32, (SUB, CH), 1)
    chunkflat = r_iota * jnp.uint32(LANE) + c_iota

    row0 = jax.lax.convert_element_type(b, jnp.uint32) * jnp.uint32(K)
    eps = jnp.finfo(jnp.float32).eps

    def gen_row(ku, dst_ref, prev_ref, rs_prev):
        base = (row0 + ku) * jnp.uint32(N) + jnp.uint32(KEY1)
        ssv = jnp.zeros((SUB, CH), jnp.float32)
        for c in range(NCH):
            sl = pl.ds(c * CH, CH)
            x1 = chunkflat + (base + jnp.uint32(c * CH))
            o0, o1 = _threefry_keyed(jnp.full((SUB, CH), KEY0, jnp.uint32), x1)
            bits = o0 ^ o1
            mant = (bits >> jnp.uint32(9)) | jnp.uint32(0x3F800000)
            u = jax.lax.bitcast_convert_type(mant, jnp.float32) - 1.0
            u = jnp.clip(u, eps, 1.0 - eps)
            w = -jnp.log(u)
            q = e8_ref[:, sl] / w
            q2 = q * q
            dst_ref[:, sl] = q2
            ssv = ssv + q2
            csamples_ref[0, :, sl] = jnp.maximum(
                csamples_ref[0, :, sl], prev_ref[:, sl] * rs_prev
            )
        return 1.0 / jnp.sum(ssv)

    def k_body(i, rs_b):
        ka = jax.lax.convert_element_type(2 * i, jnp.uint32)
        rs_a = gen_row(ka, q2a_ref, q2b_ref, rs_b)
        rs_b = gen_row(ka + jnp.uint32(1), q2b_ref, q2a_ref, rs_a)
        return rs_b

    rs_last = jax.lax.fori_loop(0, K // 2, k_body, jnp.float32(0.0))
    for c in range(NCH):
        sl = pl.ds(c * CH, CH)
        csamples_ref[0, :, sl] = jnp.maximum(
            csamples_ref[0, :, sl], q2b_ref[:, sl] * rs_last
        )


def _dsamples_kernel(logits_ref, dsamples_ref):
    l_rows = logits_ref[...]

    lb = jax.lax.bitcast_convert_type(l_rows + 0.0, jnp.uint32)
    neg = (lb >> jnp.uint32(31)) == jnp.uint32(1)
    ukey = jnp.where(neg, ~lb, lb | jnp.uint32(0x80000000))
    t = jnp.zeros((RB, 1), jnp.uint32)
    for bit in range(31, -1, -1):
        cand = t | jnp.uint32(1 << bit)
        cnt = jnp.sum((ukey >= cand).astype(jnp.int32), axis=1, keepdims=True)
        t = jnp.where(cnt >= K, cand, t)
    dsamples_ref[...] = (ukey >= t).astype(jnp.float32)


def kernel(logits):
    l_tiles = logits.reshape(B, SUB, LANE)
    spec_tile = pl.BlockSpec((1, SUB, LANE), lambda b: (b, 0, 0))
    csamples = pl.pallas_call(
        _csamples_kernel,
        grid=(B,),
        in_specs=[spec_tile],
        out_specs=spec_tile,
        out_shape=jax.ShapeDtypeStruct((B, SUB, LANE), jnp.float32),
        scratch_shapes=[
            pltpu.VMEM((SUB, LANE), jnp.float32),
            pltpu.VMEM((SUB, LANE), jnp.float32),
            pltpu.VMEM((SUB, LANE), jnp.float32),
        ],
        compiler_params=pltpu.CompilerParams(
            dimension_semantics=("parallel",),
        ),
    )(l_tiles)

    l2d = logits.reshape(B, N)
    spec_blk = pl.BlockSpec((RB, N), lambda b: (b, 0))
    dsamples = pl.pallas_call(
        _dsamples_kernel,
        grid=(B // RB,),
        in_specs=[spec_blk],
        out_specs=spec_blk,
        out_shape=jax.ShapeDtypeStruct((B, N), jnp.float32),
        compiler_params=pltpu.CompilerParams(
            dimension_semantics=("parallel",),
        ),
    )(l2d)
    return dsamples, csamples.reshape(B, N)


# speedup vs baseline: 15.1673x; 1.0001x over previous
import jax
import jax.numpy as jnp
from jax.experimental import pallas as pl
from jax.experimental.pallas import tpu as pltpu

B, K, N = 64, 16, 32768
SUB, LANE = 8, 4096
CH = 1024
NCH = LANE // CH
RB = 16

KEY0 = 928981903
KEY1 = 3453687069


def _threefry_keyed(x0, x1):
    k0 = jnp.uint32(KEY0)
    k1 = jnp.uint32(KEY1)
    k2 = jnp.uint32(KEY0 ^ KEY1 ^ 0x1BD11BDA)
    ks = (k0, k1, k2)
    rots = ((13, 15, 26, 6), (17, 29, 16, 24))

    def rotl(x, r):
        return (x << jnp.uint32(r)) | (x >> jnp.uint32(32 - r))

    for g in range(5):
        for r in rots[g % 2]:
            x0 = x0 + x1
            x1 = rotl(x1, r)
            x1 = x1 ^ x0
        x0 = x0 + ks[(g + 1) % 3]
        x1 = x1 + ks[(g + 2) % 3] + jnp.uint32(g + 1)
    return x0, x1


def _csamples_kernel(logits_ref, csamples_ref, e8_ref, q2a_ref, q2b_ref):
    b = pl.program_id(0)

    macc = jnp.full((SUB, CH), -jnp.inf, jnp.float32)
    for c in range(NCH):
        sl = pl.ds(c * CH, CH)
        macc = jnp.maximum(macc, logits_ref[0, :, sl])
    lmax = jnp.max(macc)
    zero = jnp.zeros((SUB, CH), jnp.float32)
    for c in range(NCH):
        sl = pl.ds(c * CH, CH)
        e8_ref[:, sl] = jnp.exp(logits_ref[0, :, sl] - lmax)
        csamples_ref[0, :, sl] = zero
        q2b_ref[:, sl] = zero

    r_iota = jax.lax.broadcasted_iota(jnp.uint32, (SUB, CH), 0)
    c_iota = jax.lax.broadcasted_iota(jnp.uint32, (SUB, CH), 1)
    chunkflat = r_iota * jnp.uint32(LANE) + c_iota

    row0 = jax.lax.convert_element_type(b, jnp.uint32) * jnp.uint32(K)
    eps = jnp.finfo(jnp.float32).eps

    def gen_row(ku, dst_ref, prev_ref, rs_prev):
        base = (row0 + ku) * jnp.uint32(N) + jnp.uint32(KEY1)
        ssv = jnp.zeros((SUB, CH), jnp.float32)
        for c in range(NCH):
            sl = pl.ds(c * CH, CH)
            x1 = chunkflat + (base + jnp.uint32(c * CH))
            o0, o1 = _threefry_keyed(jnp.full((SUB, CH), KEY0, jnp.uint32), x1)
            bits = o0 ^ o1
            mant = (bits >> jnp.uint32(9)) | jnp.uint32(0x3F800000)
            u = jax.lax.bitcast_convert_type(mant, jnp.float32) - 1.0
            u = jnp.clip(u, eps, 1.0 - eps)
            w = -jnp.log(u)
            q = e8_ref[:, sl] / w
            q2 = q * q
            dst_ref[:, sl] = q2
            ssv = ssv + q2
            csamples_ref[0, :, sl] = jnp.maximum(
                csamples_ref[0, :, sl], prev_ref[:, sl] * rs_prev
            )
        return 1.0 / jnp.sum(ssv)

    def k_body(i, rs_b):
        ka = jax.lax.convert_element_type(2 * i, jnp.uint32)
        rs_a = gen_row(ka, q2a_ref, q2b_ref, rs_b)
        rs_b = gen_row(ka + jnp.uint32(1), q2b_ref, q2a_ref, rs_a)
        return rs_b

    rs_last = jax.lax.fori_loop(0, K // 2, k_body, jnp.float32(0.0))
    for c in range(NCH):
        sl = pl.ds(c * CH, CH)
        csamples_ref[0, :, sl] = jnp.maximum(
            csamples_ref[0, :, sl], q2b_ref[:, sl] * rs_last
        )


def _dsamples_kernel(logits_ref, dsamples_ref):
    l_rows = logits_ref[...]

    lb = jax.lax.bitcast_convert_type(l_rows + 0.0, jnp.uint32)
    neg = (lb >> jnp.uint32(31)) == jnp.uint32(1)
    ukey = jnp.where(neg, ~lb, lb | jnp.uint32(0x80000000))
    t = jnp.zeros((RB, 1), jnp.uint32)
    for bit in range(31, -1, -1):
        cand = t | jnp.uint32(1 << bit)
        cnt = jnp.sum((ukey >= cand).astype(jnp.int32), axis=1, keepdims=True)
        t = jnp.where(cnt >= K, cand, t)
    dsamples_ref[...] = (ukey >= t).astype(jnp.float32)


def kernel(logits):
    l_tiles = logits.reshape(B, SUB, LANE)
    spec_tile = pl.BlockSpec((1, SUB, LANE), lambda b: (b, 0, 0))
    csamples = pl.pallas_call(
        _csamples_kernel,
        grid=(B,),
        in_specs=[spec_tile],
        out_specs=spec_tile,
        out_shape=jax.ShapeDtypeStruct((B, SUB, LANE), jnp.float32),
        scratch_shapes=[
            pltpu.VMEM((SUB, LANE), jnp.float32),
            pltpu.VMEM((SUB, LANE), jnp.float32),
            pltpu.VMEM((SUB, LANE), jnp.float32),
        ],
        compiler_params=pltpu.CompilerParams(
            dimension_semantics=("parallel",),
        ),
    )(l_tiles)

    l2d = logits.reshape(B, N)
    spec_blk = pl.BlockSpec((RB, N), lambda b: (b, 0))
    dsamples = pl.pallas_call(
        _dsamples_kernel,
        grid=(B // RB,),
        in_specs=[spec_blk],
        out_specs=spec_blk,
        out_shape=jax.ShapeDtypeStruct((B, N), jnp.float32),
        compiler_params=pltpu.CompilerParams(
            dimension_semantics=("parallel",),
        ),
    )(l2d)
    return dsamples, csamples.reshape(B, N)
